# baseline (speedup 1.0000x reference)
import functools

import jax
import jax.numpy as jnp
from jax import lax
from jax.experimental import pallas as pl
from jax.experimental.pallas import tpu as pltpu

B, C, HW, P = 8, 3, 224, 16
N = (HW // P) ** 2
D = 768
D_FF = 3072
H = 12
CAP = 10000
K = 5
REMAIN = 0.1
BN = B * N

MC = 2000
NEG = -3e38
BIGI = 2**30


def _ln(t):
    m = jnp.mean(t, axis=-1, keepdims=True)
    v = jnp.var(t, axis=-1, keepdims=True)
    return (t - m) / jnp.sqrt(v + 1e-6)


def _simtopk_kernel(flat_ref, mem_ref, idx_ref, qn_s, bv_s, bi_s):
    j = pl.program_id(0)

    @pl.when(j == 0)
    def _init():
        f = flat_ref[...]
        nrm = jnp.sqrt(jnp.sum(f * f, axis=1, keepdims=True)) + 1e-6
        qn_s[...] = f / nrm
        bv_s[...] = jnp.full((BN, 8), NEG, jnp.float32)
        bi_s[...] = jnp.full((BN, 8), BIGI, jnp.int32)

    mem = mem_ref[...]
    mnrm = jnp.sqrt(jnp.sum(mem * mem, axis=1, keepdims=True)) + 1e-6
    mn = mem / mnrm
    sim = lax.dot_general(qn_s[...], mn, (((1,), (1,)), ((), ())),
                          preferred_element_type=jnp.float32)
    colidx = j * MC + lax.broadcasted_iota(jnp.int32, (BN, MC), 1)
    bv = bv_s[...]
    bi = bi_s[...]
    nv, ni = [], []
    for _ in range(K):
        m = jnp.maximum(jnp.max(sim, axis=1, keepdims=True),
                        jnp.max(bv, axis=1, keepdims=True))
        i1 = jnp.min(jnp.where(sim == m, colidx, BIGI), axis=1, keepdims=True)
        i2 = jnp.min(jnp.where(bv == m, bi, BIGI), axis=1, keepdims=True)
        ii = jnp.minimum(i1, i2)
        nv.append(m)
        ni.append(ii)
        sim = jnp.where(colidx == ii, NEG, sim)
        bv = jnp.where(bi == ii, NEG, bv)
    pad_v = jnp.full((BN, 8 - K), NEG, jnp.float32)
    pad_i = jnp.full((BN, 8 - K), BIGI, jnp.int32)
    bv_s[...] = jnp.concatenate(nv + [pad_v], axis=1)
    bi_s[...] = jnp.concatenate(ni + [pad_i], axis=1)

    @pl.when(j == pl.num_programs(0) - 1)
    def _fin():
        idx_ref[...] = bi_s[...]


def _simtopk(flat, mem2):
    nsteps = CAP // MC
    return pl.pallas_call(
        _simtopk_kernel,
        grid=(nsteps,),
        in_specs=[
            pl.BlockSpec((BN, D), lambda j: (0, 0)),
            pl.BlockSpec((MC, D), lambda j: (j, 0)),
        ],
        out_specs=pl.BlockSpec((BN, 8), lambda j: (0, 0)),
        out_shape=jax.ShapeDtypeStruct((BN, 8), jnp.int32),
        scratch_shapes=[
            pltpu.VMEM((BN, D), jnp.float32),
            pltpu.VMEM((BN, 8), jnp.float32),
            pltpu.VMEM((BN, 8), jnp.int32),
        ],
    )(flat, mem2)


def _combine_kernel(flat_ref, cls_ref, nsum_ref, cm_ref, loss_ref):
    fm = jnp.mean(flat_ref[...], axis=1)
    cm = REMAIN * fm + (1.0 - REMAIN) / (K * N) * nsum_ref[...]
    cs = cls_ref[...]
    num = jnp.sum(cs * cm, axis=-1)
    den = jnp.sqrt(jnp.sum(cs * cs, axis=-1)) * jnp.sqrt(jnp.sum(cm * cm, axis=-1)) + 1e-8
    loss = jnp.mean(1.0 - num / den)
    cm_ref[...] = cm
    loss_ref[...] = jnp.full((1, 1), loss, jnp.float32)


def _combine(flat3, cls_signal, neigh_sum):
    cm, loss = pl.pallas_call(
        _combine_kernel,
        out_shape=(
            jax.ShapeDtypeStruct((B, D), jnp.float32),
            jax.ShapeDtypeStruct((1, 1), jnp.float32),
        ),
    )(flat3, cls_signal, neigh_sum)
    return cm, loss.reshape(())


def kernel(x, W_patch, b_patch, cls_tok, pos_emb, W_qkv, W_o, W_fc1, W_fc2, w_score, memory, write_idx):
    Bn = x.shape[0]
    xp = x.reshape(Bn, C, HW // P, P, HW // P, P)
    xp = xp.transpose(0, 2, 4, 1, 3, 5).reshape(Bn, N, C * P * P)
    tok = xp @ W_patch + b_patch
    tok = jnp.concatenate([jnp.broadcast_to(cls_tok, (Bn, 1, D)), tok], axis=1) + pos_emb
    h = _ln(tok)
    qkv = h @ W_qkv
    q, k, v = jnp.split(qkv, 3, axis=-1)
    dh = D // H
    def heads(t):
        return t.reshape(Bn, N + 1, H, dh).transpose(0, 2, 1, 3)
    q, k, v = heads(q), heads(k), heads(v)
    att = jax.nn.softmax((q @ k.transpose(0, 1, 3, 2)) / jnp.sqrt(float(dh)), axis=-1)
    o = (att @ v).transpose(0, 2, 1, 3).reshape(Bn, N + 1, D)
    tok = tok + o @ W_o
    tok = tok + jax.nn.gelu(_ln(tok) @ W_fc1) @ W_fc2
    cls_signal = tok[:, 0]
    flat = tok[:, 1:].reshape(Bn * N, D)
    mem2 = memory.at[write_idx].set(flat)
    nn_idx = _simtopk(flat, mem2)[:, :K]
    neigh = jnp.take(mem2, nn_idx.reshape(-1), axis=0).reshape(Bn, N, K, D)
    neigh_sum = neigh.sum(axis=(1, 2))
    return _combine(flat.reshape(Bn, N, D), cls_signal, neigh_sum)
